# Initial kernel scaffold; baseline (speedup 1.0000x reference)
import jax
import jax.numpy as jnp
from jax.experimental import pallas as pl


def kernel(inp, targ):
    raise NotImplementedError("write your pallas kernel here")


# speedup vs baseline: 1.6625x; 1.6625x over previous
import jax
import jax.numpy as jnp
from jax.experimental import pallas as pl
from jax.experimental.pallas import tpu as pltpu


def _masked_mse_block(inp_ref, targ_ref, sum_ref, cnt_ref):
    i = pl.program_id(0)

    @pl.when(i == 0)
    def _init():
        sum_ref[0] = 0.0
        cnt_ref[0] = 0.0

    t = targ_ref[...]
    x = inp_ref[...]
    mask = jnp.isnan(t)
    d = jnp.where(mask, 0.0, x - t)
    sum_ref[0] += jnp.sum(d * d)
    cnt_ref[0] += jnp.sum(jnp.where(mask, 0.0, 1.0))


def kernel(inp, targ):
    cols = inp.shape[-1]
    rows = inp.size // cols
    x = inp.reshape(rows, cols)
    t = targ.reshape(rows, cols)
    block_rows = min(512, rows)
    grid = rows // block_rows
    s, c = pl.pallas_call(
        _masked_mse_block,
        grid=(grid,),
        in_specs=[
            pl.BlockSpec((block_rows, cols), lambda i: (i, 0)),
            pl.BlockSpec((block_rows, cols), lambda i: (i, 0)),
        ],
        out_specs=[
            pl.BlockSpec(memory_space=pltpu.SMEM),
            pl.BlockSpec(memory_space=pltpu.SMEM),
        ],
        out_shape=[
            jax.ShapeDtypeStruct((1,), jnp.float32),
            jax.ShapeDtypeStruct((1,), jnp.float32),
        ],
    )(x, t)
    return s[0] / c[0]


# speedup vs baseline: 1.6636x; 1.0007x over previous
import jax
import jax.numpy as jnp
from jax.experimental import pallas as pl
from jax.experimental.pallas import tpu as pltpu


def _masked_mse_block(inp_ref, targ_ref, sum_ref, cnt_ref):
    i = pl.program_id(0)

    @pl.when(i == 0)
    def _init():
        sum_ref[0] = 0.0
        cnt_ref[0] = 0.0

    t = targ_ref[...]
    x = inp_ref[...]
    mask = jnp.isnan(t)
    d = jnp.where(mask, 0.0, x - t)
    sum_ref[0] += jnp.sum(d * d)
    cnt_ref[0] += jnp.sum(jnp.where(mask, 0.0, 1.0))


def kernel(inp, targ):
    cols = inp.shape[-1]
    rows = inp.size // cols
    x = inp.reshape(rows, cols)
    t = targ.reshape(rows, cols)
    block_rows = min(1024, rows)
    grid = rows // block_rows
    s, c = pl.pallas_call(
        _masked_mse_block,
        grid=(grid,),
        in_specs=[
            pl.BlockSpec((block_rows, cols), lambda i: (i, 0)),
            pl.BlockSpec((block_rows, cols), lambda i: (i, 0)),
        ],
        out_specs=[
            pl.BlockSpec(memory_space=pltpu.SMEM),
            pl.BlockSpec(memory_space=pltpu.SMEM),
        ],
        out_shape=[
            jax.ShapeDtypeStruct((1,), jnp.float32),
            jax.ShapeDtypeStruct((1,), jnp.float32),
        ],
    )(x, t)
    return s[0] / c[0]
